# baseline (speedup 1.0000x reference)
import functools
import math

import jax
import jax.numpy as jnp
from jax import lax
from jax.experimental import pallas as pl
from jax.experimental.pallas import tpu as pltpu
from jax.experimental.pallas import tpu_sc as plsc

EMBED_DIM = 128
ENGRAM_DIM = 64
B = 4
T = 4096
BT = B * T
N_STREAMS = 4
NW = 32
PAD_ROW = T + 128
EPS = float(jnp.finfo(jnp.float32).eps)
TB = 1024
GC = N_STREAMS * EMBED_DIM

N_HALF = 4
TOK_H = BT // N_HALF
TOK_W = TOK_H // NW
N_BLOCKS = TOK_H // TB


def _sc_hash_gather(ids_pad_flat, mults_bc, tab128, half):
    mesh = plsc.VectorSubcoreMesh(core_axis_name="c", subcore_axis_name="s")

    grp = TOK_W // 16
    n_chunk = TOK_W // 128
    rows_per_b = T // TOK_W

    @functools.partial(
        pl.kernel,
        mesh=mesh,
        out_type=jax.ShapeDtypeStruct((TOK_H, 256), jnp.float32),
        scratch_types=[
            pltpu.VMEM((TOK_W + 128,), jnp.int32),
            pltpu.VMEM((16, 128), jnp.int32),
            pltpu.VMEM((16, 128), jnp.int32),
            pltpu.VMEM((128, 128), jnp.float32),
            pltpu.VMEM((128, 128), jnp.float32),
            pltpu.VMEM_SHARED((16, 128, 128), jnp.float32),
            pltpu.SemaphoreType.DMA,
            pltpu.SemaphoreType.DMA,
        ],
    )
    def sc_kernel(ids_hbm, m_hbm, tab_hbm, ec,
                  ids_v, m_v, idx_v, rows_a, rows_b, pk, sem_a, sem_b):
        wid = lax.axis_index("s") * 2 + lax.axis_index("c")
        b = half * (B // N_HALF) + wid // rows_per_b
        lt = (wid % rows_per_b) * TOK_W
        off = b * PAD_ROW + lt
        pltpu.sync_copy(ids_hbm.at[pl.ds(off, TOK_W + 128)], ids_v)
        pltpu.sync_copy(m_hbm, m_v)

        for i in range(grp):
            cur = ids_v[pl.ds(2 + 16 * i, 16)]
            p1 = ids_v[pl.ds(1 + 16 * i, 16)]
            p2 = ids_v[pl.ds(16 * i, 16)]
            m = [m_v[r, pl.ds(0, 16)] for r in range(10)]
            h0 = ((p1 * m[0]) ^ (cur * m[1])) & 4095
            h1 = (((p1 * m[2]) ^ (cur * m[3])) & 4095) + 4096
            h2 = (((p2 * m[4]) ^ (p1 * m[5]) ^ (cur * m[6])) & 4095) + 8192
            h3 = (((p2 * m[7]) ^ (p1 * m[8]) ^ (cur * m[9])) & 4095) + 12288
            c, col = i // 8, (i % 8) * 16
            idx_v[0 * n_chunk + c, pl.ds(col, 16)] = h0
            idx_v[1 * n_chunk + c, pl.ds(col, 16)] = h1
            idx_v[2 * n_chunk + c, pl.ds(col, 16)] = h2
            idx_v[3 * n_chunk + c, pl.ds(col, 16)] = h3

        for c in range(n_chunk):
            for kk in range(2):
                cpa = pltpu.async_copy(
                    tab_hbm.at[idx_v.at[(2 * kk) * n_chunk + c]], rows_a,
                    sem_a)
                cpb = pltpu.async_copy(
                    tab_hbm.at[idx_v.at[(2 * kk + 1) * n_chunk + c]], rows_b,
                    sem_b)
                sid = lax.axis_index("s")
                cpa.wait()
                pltpu.sync_copy(rows_a.at[:, pl.ds(0, 64)],
                                pk.at[sid, :, pl.ds(0, 64)])
                cpb.wait()
                pltpu.sync_copy(rows_b.at[:, pl.ds(0, 64)],
                                pk.at[sid, :, pl.ds(64, 64)])
                pltpu.sync_copy(
                    pk.at[sid],
                    ec.at[pl.ds(wid * TOK_W + c * 128, 128),
                          pl.ds(kk * 128, 128)])

    return sc_kernel(ids_pad_flat, mults_bc, tab128)


def _make_tc_body(half, has_y):
    base = half * TOK_H

    def _tc_body(ec, x_any, wc, bc, nq, nk, scn, cw, *rest):
        rest = rest[1:] if has_y else rest
        out_any, xscr, xq, ybuf, sem_i, sem_o = rest
        p = pl.program_id(0)
        par = p % 2
        rows = pl.ds(base + p * TB, TB)

        nxt = jnp.minimum(p + 1, N_BLOCKS - 1)
        nrows = pl.ds(base + nxt * TB, TB)

        @pl.when(p == 0)
        def _():
            for i in range(N_STREAMS):
                pltpu.make_async_copy(
                    x_any.at[rows, i], xq.at[0, i], sem_i.at[0]).start()

        for i in range(N_STREAMS):
            pltpu.make_async_copy(
                x_any.at[nrows, i], xq.at[1 - par, i],
                sem_i.at[1 - par]).start()

        h = (jnp.dot(ec[...], wc[...], preferred_element_type=jnp.float32)
             + bc[...])
        vb = h[:, :EMBED_DIM]
        inv_sqrt_d = 1.0 / math.sqrt(EMBED_DIM)

        for i in range(N_STREAMS):
            pltpu.make_async_copy(
                x_any.at[rows, i], xq.at[par, i], sem_i.at[par]).wait()

        @pl.when(p >= 2)
        def _():
            for i in range(N_STREAMS):
                pltpu.make_async_copy(
                    ybuf.at[par, i], out_any.at[rows, i],
                    sem_o.at[par]).wait()

        start = (p % (T // TB)) == 0
        for i in range(N_STREAMS):
            k = h[:, EMBED_DIM * (i + 1):EMBED_DIM * (i + 2)]
            q = xq[par, i]
            qn = (q * lax.rsqrt(jnp.mean(q * q, axis=1, keepdims=True) + EPS)
                  * nq[i])
            kn = (k * lax.rsqrt(jnp.mean(k * k, axis=1, keepdims=True) + EPS)
                  * nk[i])
            s = jnp.sum(qn * kn, axis=1, keepdims=True) * inv_sqrt_d
            g = jax.nn.sigmoid(s)
            vg = vb * g
            xn = (vg * lax.rsqrt(jnp.mean(vg * vg, axis=1, keepdims=True)
                                 + EPS) * scn[i])

            tail = xscr[i, 8 + TB - 3:8 + TB, :]
            xscr[i, 5:8, :] = jnp.where(start, 0.0, tail)
            xscr[i, 8:8 + TB, :] = xn
            y = (cw[0, i] * xscr[i, 5:5 + TB, :]
                 + cw[1, i] * xscr[i, 6:6 + TB, :]
                 + cw[2, i] * xscr[i, 7:7 + TB, :]
                 + cw[3, i] * xscr[i, 8:8 + TB, :])
            ybuf[par, i] = vg + y * jax.nn.sigmoid(y)

        cps_o = []
        for i in range(N_STREAMS):
            cp = pltpu.make_async_copy(
                ybuf.at[par, i], out_any.at[rows, i], sem_o.at[par])
            cp.start()
            cps_o.append(cp)

        @pl.when(p == N_BLOCKS - 1)
        def _():
            for cp in cps_o:
                cp.wait()
            for i in range(N_STREAMS):
                pltpu.make_async_copy(
                    ybuf.at[1 - par, i], out_any.at[rows, i],
                    sem_o.at[1 - par]).wait()
            for i in range(N_STREAMS):
                pltpu.make_async_copy(
                    x_any.at[rows, i], xq.at[1 - par, i],
                    sem_i.at[1 - par]).wait()

    return _tc_body


def _tc_stage(ec, x3, wc, bc, nq, nk, scn, cw, y_in, half):
    full = lambda shape: pl.BlockSpec(shape, lambda p: (0,) * len(shape))
    has_y = y_in is not None
    in_specs = [
        pl.BlockSpec((TB, 256), lambda p: (p, 0)),
        pl.BlockSpec(memory_space=pl.ANY),
        full((256, 640)),
        full((1, 640)),
        full((N_STREAMS, EMBED_DIM)),
        full((N_STREAMS, EMBED_DIM)),
        full((N_STREAMS, EMBED_DIM)),
        full((4, N_STREAMS, EMBED_DIM)),
    ]
    args = [ec, x3, wc, bc, nq, nk, scn, cw]
    if has_y:
        in_specs.append(pl.BlockSpec(memory_space=pl.ANY))
        args.append(y_in)
    return pl.pallas_call(
        _make_tc_body(half, has_y),
        grid=(N_BLOCKS,),
        in_specs=in_specs,
        out_specs=pl.BlockSpec(memory_space=pl.ANY),
        out_shape=jax.ShapeDtypeStruct((BT, N_STREAMS, EMBED_DIM),
                                       jnp.float32),
        input_output_aliases={8: 0} if has_y else {},
        scratch_shapes=[
            pltpu.VMEM((N_STREAMS, TB + 8, EMBED_DIM), jnp.float32),
            pltpu.VMEM((2, N_STREAMS, TB, EMBED_DIM), jnp.float32),
            pltpu.VMEM((2, N_STREAMS, TB, EMBED_DIM), jnp.float32),
            pltpu.SemaphoreType.DMA((2,)),
            pltpu.SemaphoreType.DMA((2,)),
        ],
        compiler_params=pltpu.CompilerParams(
            dimension_semantics=("arbitrary",)),
    )(*args)


def kernel(x, input_ids, multipliers, emb_table, val_W, val_b, key_W, key_b,
           nq_w, nk_w, conv_w, sc_norm_w):
    ids_pad = jnp.pad(input_ids, ((0, 0), (2, PAD_ROW - T - 2))).reshape(-1)
    mflat = jnp.concatenate(
        [multipliers[0, :, :2].reshape(-1), multipliers[1, :, :3].reshape(-1),
         jnp.zeros((6,), multipliers.dtype)])
    mbc = jnp.broadcast_to(mflat[:, None], (16, 128))
    tab128 = jnp.pad(emb_table, ((0, 0), (0, 128 - ENGRAM_DIM)))

    x3 = x.reshape(BT, N_STREAMS, EMBED_DIM)
    wc = jnp.concatenate([val_W.T] + [key_W[i].T for i in range(N_STREAMS)],
                         axis=1)
    bc = jnp.concatenate([val_b, key_b.reshape(-1)])[None, :]
    cw = jnp.transpose(conv_w.reshape(N_STREAMS, EMBED_DIM, 4), (2, 0, 1))

    ecs = [_sc_hash_gather(ids_pad, mbc, tab128, q) for q in range(N_HALF)]
    y = None
    for q in range(N_HALF):
        y = _tc_stage(ecs[q], x3, wc, bc, nq_w, nk_w, sc_norm_w, cw, y, q)
    return y.reshape(B, T, N_STREAMS, EMBED_DIM)


# speedup vs baseline: 1.0280x; 1.0280x over previous
import functools
import math

import jax
import jax.numpy as jnp
from jax import lax
from jax.experimental import pallas as pl
from jax.experimental.pallas import tpu as pltpu
from jax.experimental.pallas import tpu_sc as plsc

EMBED_DIM = 128
ENGRAM_DIM = 64
B = 4
T = 4096
BT = B * T
N_STREAMS = 4
NW = 32
PAD_ROW = T + 128
EPS = float(jnp.finfo(jnp.float32).eps)
TB = 1024
GC = N_STREAMS * EMBED_DIM

N_HALF = 2
TOK_H = BT // N_HALF
TOK_W = TOK_H // NW
N_BLOCKS = TOK_H // TB


def _sc_hash_gather(ids_pad_flat, mults_bc, tab128, half):
    mesh = plsc.VectorSubcoreMesh(core_axis_name="c", subcore_axis_name="s")

    grp = TOK_W // 16
    n_chunk = TOK_W // 128
    rows_per_b = T // TOK_W

    @functools.partial(
        pl.kernel,
        mesh=mesh,
        out_type=jax.ShapeDtypeStruct((TOK_H, 256), jnp.float32),
        scratch_types=[
            pltpu.VMEM((TOK_W + 128,), jnp.int32),
            pltpu.VMEM((16, 128), jnp.int32),
            pltpu.VMEM((16, 128), jnp.int32),
            pltpu.VMEM((128, 128), jnp.float32),
            pltpu.VMEM((128, 128), jnp.float32),
            pltpu.VMEM_SHARED((16, 128, 128), jnp.float32),
            pltpu.SemaphoreType.DMA,
            pltpu.SemaphoreType.DMA,
        ],
    )
    def sc_kernel(ids_hbm, m_hbm, tab_hbm, ec,
                  ids_v, m_v, idx_v, rows_a, rows_b, pk, sem_a, sem_b):
        wid = lax.axis_index("s") * 2 + lax.axis_index("c")
        b = half * (B // N_HALF) + wid // rows_per_b
        lt = (wid % rows_per_b) * TOK_W
        off = b * PAD_ROW + lt
        pltpu.sync_copy(ids_hbm.at[pl.ds(off, TOK_W + 128)], ids_v)
        pltpu.sync_copy(m_hbm, m_v)

        for i in range(grp):
            cur = ids_v[pl.ds(2 + 16 * i, 16)]
            p1 = ids_v[pl.ds(1 + 16 * i, 16)]
            p2 = ids_v[pl.ds(16 * i, 16)]
            m = [m_v[r, pl.ds(0, 16)] for r in range(10)]
            h0 = ((p1 * m[0]) ^ (cur * m[1])) & 4095
            h1 = (((p1 * m[2]) ^ (cur * m[3])) & 4095) + 4096
            h2 = (((p2 * m[4]) ^ (p1 * m[5]) ^ (cur * m[6])) & 4095) + 8192
            h3 = (((p2 * m[7]) ^ (p1 * m[8]) ^ (cur * m[9])) & 4095) + 12288
            c, col = i // 8, (i % 8) * 16
            idx_v[0 * n_chunk + c, pl.ds(col, 16)] = h0
            idx_v[1 * n_chunk + c, pl.ds(col, 16)] = h1
            idx_v[2 * n_chunk + c, pl.ds(col, 16)] = h2
            idx_v[3 * n_chunk + c, pl.ds(col, 16)] = h3

        for c in range(n_chunk):
            for kk in range(2):
                cpa = pltpu.async_copy(
                    tab_hbm.at[idx_v.at[(2 * kk) * n_chunk + c]], rows_a,
                    sem_a)
                cpb = pltpu.async_copy(
                    tab_hbm.at[idx_v.at[(2 * kk + 1) * n_chunk + c]], rows_b,
                    sem_b)
                sid = lax.axis_index("s")
                cpa.wait()
                pltpu.sync_copy(rows_a.at[:, pl.ds(0, 64)],
                                pk.at[sid, :, pl.ds(0, 64)])
                cpb.wait()
                pltpu.sync_copy(rows_b.at[:, pl.ds(0, 64)],
                                pk.at[sid, :, pl.ds(64, 64)])
                pltpu.sync_copy(
                    pk.at[sid],
                    ec.at[pl.ds(wid * TOK_W + c * 128, 128),
                          pl.ds(kk * 128, 128)])

    return sc_kernel(ids_pad_flat, mults_bc, tab128)


def _make_tc_body(half, has_y):
    base = half * TOK_H

    def _tc_body(ec, x_any, wc, bc, nq, nk, scn, cw, *rest):
        rest = rest[1:] if has_y else rest
        out_any, xscr, xq, ybuf, sem_i, sem_o = rest
        p = pl.program_id(0)
        par = p % 2
        rows = pl.ds(base + p * TB, TB)

        nxt = jnp.minimum(p + 1, N_BLOCKS - 1)
        nrows = pl.ds(base + nxt * TB, TB)

        @pl.when(p == 0)
        def _():
            for i in range(N_STREAMS):
                pltpu.make_async_copy(
                    x_any.at[rows, i], xq.at[0, i], sem_i.at[0]).start()

        for i in range(N_STREAMS):
            pltpu.make_async_copy(
                x_any.at[nrows, i], xq.at[1 - par, i],
                sem_i.at[1 - par]).start()

        h = (jnp.dot(ec[...], wc[...], preferred_element_type=jnp.float32)
             + bc[...])
        vb = h[:, :EMBED_DIM]
        inv_sqrt_d = 1.0 / math.sqrt(EMBED_DIM)

        for i in range(N_STREAMS):
            pltpu.make_async_copy(
                x_any.at[rows, i], xq.at[par, i], sem_i.at[par]).wait()

        @pl.when(p >= 2)
        def _():
            for i in range(N_STREAMS):
                pltpu.make_async_copy(
                    ybuf.at[par, i], out_any.at[rows, i],
                    sem_o.at[par]).wait()

        start = (p % (T // TB)) == 0
        for i in range(N_STREAMS):
            k = h[:, EMBED_DIM * (i + 1):EMBED_DIM * (i + 2)]
            q = xq[par, i]
            qn = (q * lax.rsqrt(jnp.mean(q * q, axis=1, keepdims=True) + EPS)
                  * nq[i])
            kn = (k * lax.rsqrt(jnp.mean(k * k, axis=1, keepdims=True) + EPS)
                  * nk[i])
            s = jnp.sum(qn * kn, axis=1, keepdims=True) * inv_sqrt_d
            g = jax.nn.sigmoid(s)
            vg = vb * g
            xn = (vg * lax.rsqrt(jnp.mean(vg * vg, axis=1, keepdims=True)
                                 + EPS) * scn[i])

            tail = xscr[i, 8 + TB - 3:8 + TB, :]
            xscr[i, 5:8, :] = jnp.where(start, 0.0, tail)
            xscr[i, 8:8 + TB, :] = xn
            y = (cw[0, i] * xscr[i, 5:5 + TB, :]
                 + cw[1, i] * xscr[i, 6:6 + TB, :]
                 + cw[2, i] * xscr[i, 7:7 + TB, :]
                 + cw[3, i] * xscr[i, 8:8 + TB, :])
            ybuf[par, i] = vg + y * jax.nn.sigmoid(y)

        cps_o = []
        for i in range(N_STREAMS):
            cp = pltpu.make_async_copy(
                ybuf.at[par, i], out_any.at[rows, i], sem_o.at[par])
            cp.start()
            cps_o.append(cp)

        @pl.when(p == N_BLOCKS - 1)
        def _():
            for cp in cps_o:
                cp.wait()
            for i in range(N_STREAMS):
                pltpu.make_async_copy(
                    ybuf.at[1 - par, i], out_any.at[rows, i],
                    sem_o.at[1 - par]).wait()
            for i in range(N_STREAMS):
                pltpu.make_async_copy(
                    x_any.at[rows, i], xq.at[1 - par, i],
                    sem_i.at[1 - par]).wait()

    return _tc_body


def _tc_stage(ec, x3, wc, bc, nq, nk, scn, cw, y_in, half):
    full = lambda shape: pl.BlockSpec(shape, lambda p: (0,) * len(shape))
    has_y = y_in is not None
    in_specs = [
        pl.BlockSpec((TB, 256), lambda p: (p, 0)),
        pl.BlockSpec(memory_space=pl.ANY),
        full((256, 640)),
        full((1, 640)),
        full((N_STREAMS, EMBED_DIM)),
        full((N_STREAMS, EMBED_DIM)),
        full((N_STREAMS, EMBED_DIM)),
        full((4, N_STREAMS, EMBED_DIM)),
    ]
    args = [ec, x3, wc, bc, nq, nk, scn, cw]
    if has_y:
        in_specs.append(pl.BlockSpec(memory_space=pl.ANY))
        args.append(y_in)
    return pl.pallas_call(
        _make_tc_body(half, has_y),
        grid=(N_BLOCKS,),
        in_specs=in_specs,
        out_specs=pl.BlockSpec(memory_space=pl.ANY),
        out_shape=jax.ShapeDtypeStruct((BT, N_STREAMS, EMBED_DIM),
                                       jnp.float32),
        input_output_aliases={8: 0} if has_y else {},
        scratch_shapes=[
            pltpu.VMEM((N_STREAMS, TB + 8, EMBED_DIM), jnp.float32),
            pltpu.VMEM((2, N_STREAMS, TB, EMBED_DIM), jnp.float32),
            pltpu.VMEM((2, N_STREAMS, TB, EMBED_DIM), jnp.float32),
            pltpu.SemaphoreType.DMA((2,)),
            pltpu.SemaphoreType.DMA((2,)),
        ],
        compiler_params=pltpu.CompilerParams(
            dimension_semantics=("arbitrary",)),
    )(*args)


def kernel(x, input_ids, multipliers, emb_table, val_W, val_b, key_W, key_b,
           nq_w, nk_w, conv_w, sc_norm_w):
    ids_pad = jnp.pad(input_ids, ((0, 0), (2, PAD_ROW - T - 2))).reshape(-1)
    mflat = jnp.concatenate(
        [multipliers[0, :, :2].reshape(-1), multipliers[1, :, :3].reshape(-1),
         jnp.zeros((6,), multipliers.dtype)])
    mbc = jnp.broadcast_to(mflat[:, None], (16, 128))
    tab128 = jnp.pad(emb_table, ((0, 0), (0, 128 - ENGRAM_DIM)))

    x3 = x.reshape(BT, N_STREAMS, EMBED_DIM)
    wc = jnp.concatenate([val_W.T] + [key_W[i].T for i in range(N_STREAMS)],
                         axis=1)
    bc = jnp.concatenate([val_b, key_b.reshape(-1)])[None, :]
    cw = jnp.transpose(conv_w.reshape(N_STREAMS, EMBED_DIM, 4), (2, 0, 1))

    ecs = [_sc_hash_gather(ids_pad, mbc, tab128, q) for q in range(N_HALF)]
    y = None
    for q in range(N_HALF):
        y = _tc_stage(ecs[q], x3, wc, bc, nq_w, nk_w, sc_norm_w, cw, y, q)
    return y.reshape(B, T, N_STREAMS, EMBED_DIM)
